# baseline (speedup 1.0000x reference)
import functools

import jax
import jax.numpy as jnp
from jax import lax
from jax.experimental import pallas as pl
from jax.experimental.pallas import tpu as pltpu
from jax.experimental.pallas import tpu_sc as plsc

NSEG = 512
N = 320000
D = 128
DV = D // 16

NW = 32
ROWS_W = N // NW
C = 80
SB = 80
NSB = C // SB
NCHUNK = ROWS_W // C
NIDS = ROWS_W // SB
ZR = NSEG // 16


def _sc_body(x_hbm, b2d_hbm, out_hbm, xbuf, ids, zbuf, shared, sems, ssems):
    cid = lax.axis_index("c")
    sid = lax.axis_index("s")
    wid = sid * 2 + cid
    base = wid * ROWS_W

    def dma_x(k, s):
        return pltpu.make_async_copy(
            x_hbm.at[pl.ds(base + k * C, C)], xbuf.at[s], sems.at[s]
        )

    def scat(k, s):
        return pltpu.make_async_copy(
            xbuf.at[s], shared.at[ids.at[k]], ssems.at[s]
        )

    idcp = pltpu.make_async_copy(b2d_hbm.at[wid], ids, sems.at[4])
    idcp.start()

    zero = jnp.zeros((16,), jnp.float32)

    def zrow(r, carry):
        row = zbuf.at[r]
        for j in range(DV):
            row[pl.ds(16 * j, 16)] = zero
        return carry

    lax.fori_loop(0, ZR, zrow, 0)
    pltpu.sync_copy(zbuf, shared.at[pl.ds(sid * ZR, ZR)])
    plsc.subcore_barrier()

    for s in range(4):
        dma_x(s, s).start()
    idcp.wait()

    def quad(q, carry):
        for s in range(4):
            k = 4 * q + s
            dma_x(k, s).wait()
            scat(k, s).start(add=True)
        for s in range(4):
            k = 4 * q + s
            scat(k, s).wait()

            @pl.when(k + 4 < NCHUNK)
            def _(k=k, s=s):
                dma_x(k + 4, s).start()

        return carry

    lax.fori_loop(0, NCHUNK // 4, quad, 0)
    kt = NCHUNK - 1
    dma_x(kt, 0).wait()
    scat(kt, 0).start(add=True)
    scat(kt, 0).wait()

    plsc.subcore_barrier()
    pltpu.sync_copy(
        shared.at[pl.ds(sid * ZR, ZR)],
        out_hbm.at[cid].at[pl.ds(sid * ZR, ZR)],
    )


def _combine_body(p_ref, o_ref):
    o_ref[...] = p_ref[0] + p_ref[1]


def kernel(x, batch):
    b2d = batch.astype(jnp.int32).reshape(NW, NIDS, SB)
    sc = pl.kernel(
        _sc_body,
        out_type=jax.ShapeDtypeStruct((2, NSEG, D), jnp.float32),
        mesh=plsc.VectorSubcoreMesh(core_axis_name="c", subcore_axis_name="s"),
        scratch_types=[
            pltpu.VMEM((4, C, D), jnp.float32),
            pltpu.VMEM((NIDS, SB), jnp.int32),
            pltpu.VMEM((ZR, D), jnp.float32),
            pltpu.VMEM_SHARED((NSEG, D), jnp.float32),
            pltpu.SemaphoreType.DMA((5,)),
            pltpu.SemaphoreType.DMA((4,)),
        ],
    )
    partials = sc(x, b2d)
    out = pl.pallas_call(
        _combine_body,
        out_shape=jax.ShapeDtypeStruct((NSEG, D), jnp.float32),
    )(partials)
    return out


# speedup vs baseline: 1.3785x; 1.3785x over previous
import functools

import jax
import jax.numpy as jnp
from jax import lax
from jax.experimental import pallas as pl
from jax.experimental.pallas import tpu as pltpu
from jax.experimental.pallas import tpu_sc as plsc

NSEG = 512
N = 320000
D = 128
DV = D // 16

R0 = 115200

BLK = 1280
NBLK = R0 // BLK

NW = 32
NSC = N - R0
ROWS_W = NSC // NW
C = 80
NCHUNK = ROWS_W // C
ZR = NSEG // 16


def _sc_body(x_hbm, b2d_hbm, out_hbm, xbuf, ids, zbuf, shared, sems, ssems):
    cid = lax.axis_index("c")
    sid = lax.axis_index("s")
    wid = sid * 2 + cid
    base = R0 + wid * ROWS_W

    def dma_x(k, s):
        return pltpu.make_async_copy(
            x_hbm.at[pl.ds(base + k * C, C)], xbuf.at[s], sems.at[s]
        )

    def scat(k, s):
        return pltpu.make_async_copy(
            xbuf.at[s], shared.at[ids.at[k]], ssems.at[s]
        )

    idcp = pltpu.make_async_copy(b2d_hbm.at[wid], ids, sems.at[4])
    idcp.start()

    zero = jnp.zeros((16,), jnp.float32)

    def zrow(r, carry):
        row = zbuf.at[r]
        for j in range(DV):
            row[pl.ds(16 * j, 16)] = zero
        return carry

    lax.fori_loop(0, ZR, zrow, 0)
    pltpu.sync_copy(zbuf, shared.at[pl.ds(sid * ZR, ZR)])
    plsc.subcore_barrier()

    for s in range(4):
        dma_x(s, s).start()
    idcp.wait()

    def quad(q, carry):
        for s in range(4):
            k = 4 * q + s
            dma_x(k, s).wait()
            scat(k, s).start(add=True)
        for s in range(4):
            k = 4 * q + s
            scat(k, s).wait()

            @pl.when(k + 4 < NCHUNK)
            def _(k=k, s=s):
                dma_x(k + 4, s).start()

        return carry

    lax.fori_loop(0, NCHUNK // 4, quad, 0)

    plsc.subcore_barrier()
    pltpu.sync_copy(
        shared.at[pl.ds(sid * ZR, ZR)],
        out_hbm.at[cid].at[pl.ds(sid * ZR, ZR)],
    )


def _tc_body(batch_ref, x_ref, out_ref):
    i = pl.program_id(0)
    b = batch_ref[0, 0, :]
    onehot = (
        jax.lax.broadcasted_iota(jnp.int32, (NSEG, BLK), 0) == b[None, :]
    ).astype(jnp.float32)
    part = jax.lax.dot_general(
        onehot, x_ref[...], (((1,), (0,)), ((), ())),
        preferred_element_type=jnp.float32,
    )

    @pl.when(i == 0)
    def _():
        out_ref[...] = part

    @pl.when(i > 0)
    def _():
        out_ref[...] += part


def _combine_body(t_ref, p_ref, o_ref):
    o_ref[...] = t_ref[...] + p_ref[0] + p_ref[1]


def kernel(x, batch):
    b32 = batch.astype(jnp.int32)
    b2d = b32[R0:].reshape(NW, NCHUNK, C)
    batch3 = b32.reshape(N // BLK, 1, BLK)

    sc = pl.kernel(
        _sc_body,
        out_type=jax.ShapeDtypeStruct((2, NSEG, D), jnp.float32),
        mesh=plsc.VectorSubcoreMesh(core_axis_name="c", subcore_axis_name="s"),
        scratch_types=[
            pltpu.VMEM((4, C, D), jnp.float32),
            pltpu.VMEM((NCHUNK, C), jnp.int32),
            pltpu.VMEM((ZR, D), jnp.float32),
            pltpu.VMEM_SHARED((NSEG, D), jnp.float32),
            pltpu.SemaphoreType.DMA((5,)),
            pltpu.SemaphoreType.DMA((4,)),
        ],
    )
    partials = sc(x, b2d)

    tcp = pl.pallas_call(
        _tc_body,
        grid=(NBLK,),
        in_specs=[
            pl.BlockSpec((1, 1, BLK), lambda i: (i, 0, 0)),
            pl.BlockSpec((BLK, D), lambda i: (i, 0)),
        ],
        out_specs=pl.BlockSpec((NSEG, D), lambda i: (0, 0)),
        out_shape=jax.ShapeDtypeStruct((NSEG, D), jnp.float32),
    )(batch3, x)

    out = pl.pallas_call(
        _combine_body,
        out_shape=jax.ShapeDtypeStruct((NSEG, D), jnp.float32),
    )(tcp, partials)
    return out


# speedup vs baseline: 1.5007x; 1.0887x over previous
import functools

import jax
import jax.numpy as jnp
from jax import lax
from jax.experimental import pallas as pl
from jax.experimental.pallas import tpu as pltpu
from jax.experimental.pallas import tpu_sc as plsc

NSEG = 512
N = 320000
D = 128
DV = D // 16

R0 = 74240

BLK = 1280
NBLK = R0 // BLK

NW = 32
NSC = N - R0
ROWS_W = NSC // NW
C = 80
NCHUNK = ROWS_W // C
ZR = NSEG // 16


def _sc_body(x_hbm, b2d_hbm, out_hbm, xbuf, ids, zbuf, shared, sems, ssems):
    cid = lax.axis_index("c")
    sid = lax.axis_index("s")
    wid = sid * 2 + cid
    base = R0 + wid * ROWS_W

    def dma_x(k, s):
        return pltpu.make_async_copy(
            x_hbm.at[pl.ds(base + k * C, C)], xbuf.at[s], sems.at[s]
        )

    def scat(k, s):
        return pltpu.make_async_copy(
            xbuf.at[s], shared.at[ids.at[k]], ssems.at[s]
        )

    idcp = pltpu.make_async_copy(b2d_hbm.at[wid], ids, sems.at[4])
    idcp.start()

    zero = jnp.zeros((16,), jnp.float32)

    def zrow(r, carry):
        row = zbuf.at[r]
        for j in range(DV):
            row[pl.ds(16 * j, 16)] = zero
        return carry

    lax.fori_loop(0, ZR, zrow, 0)
    pltpu.sync_copy(zbuf, shared.at[pl.ds(sid * ZR, ZR)])
    plsc.subcore_barrier()

    for s in range(4):
        dma_x(s, s).start()
    idcp.wait()

    def quad(q, carry):
        for s in range(4):
            k = 4 * q + s
            dma_x(k, s).wait()
            scat(k, s).start(add=True)
        for s in range(4):
            k = 4 * q + s
            scat(k, s).wait()

            @pl.when(k + 4 < NCHUNK)
            def _(k=k, s=s):
                dma_x(k + 4, s).start()

        return carry

    lax.fori_loop(0, NCHUNK // 4, quad, 0)

    plsc.subcore_barrier()
    pltpu.sync_copy(
        shared.at[pl.ds(sid * ZR, ZR)],
        out_hbm.at[cid].at[pl.ds(sid * ZR, ZR)],
    )


def _tc_body(batch_ref, x_ref, out_ref):
    i = pl.program_id(0)
    b = batch_ref[0, 0, :]
    onehot = (
        jax.lax.broadcasted_iota(jnp.int32, (NSEG, BLK), 0) == b[None, :]
    ).astype(jnp.float32)
    part = jax.lax.dot_general(
        onehot, x_ref[...], (((1,), (0,)), ((), ())),
        preferred_element_type=jnp.float32,
    )

    @pl.when(i == 0)
    def _():
        out_ref[...] = part

    @pl.when(i > 0)
    def _():
        out_ref[...] += part


def _combine_body(t_ref, p_ref, o_ref):
    o_ref[...] = t_ref[...] + p_ref[0] + p_ref[1]


def kernel(x, batch):
    b32 = batch.astype(jnp.int32)
    b2d = b32[R0:].reshape(NW, NCHUNK, C)
    batch3 = b32.reshape(N // BLK, 1, BLK)

    sc = pl.kernel(
        _sc_body,
        out_type=jax.ShapeDtypeStruct((2, NSEG, D), jnp.float32),
        mesh=plsc.VectorSubcoreMesh(core_axis_name="c", subcore_axis_name="s"),
        scratch_types=[
            pltpu.VMEM((4, C, D), jnp.float32),
            pltpu.VMEM((NCHUNK, C), jnp.int32),
            pltpu.VMEM((ZR, D), jnp.float32),
            pltpu.VMEM_SHARED((NSEG, D), jnp.float32),
            pltpu.SemaphoreType.DMA((5,)),
            pltpu.SemaphoreType.DMA((4,)),
        ],
    )
    partials = sc(x, b2d)

    tcp = pl.pallas_call(
        _tc_body,
        grid=(NBLK,),
        in_specs=[
            pl.BlockSpec((1, 1, BLK), lambda i: (i, 0, 0)),
            pl.BlockSpec((BLK, D), lambda i: (i, 0)),
        ],
        out_specs=pl.BlockSpec((NSEG, D), lambda i: (0, 0)),
        out_shape=jax.ShapeDtypeStruct((NSEG, D), jnp.float32),
    )(batch3, x)

    out = pl.pallas_call(
        _combine_body,
        out_shape=jax.ShapeDtypeStruct((NSEG, D), jnp.float32),
    )(tcp, partials)
    return out


# speedup vs baseline: 1.5098x; 1.0061x over previous
import functools

import jax
import jax.numpy as jnp
from jax import lax
from jax.experimental import pallas as pl
from jax.experimental.pallas import tpu as pltpu
from jax.experimental.pallas import tpu_sc as plsc

NSEG = 512
N = 320000
D = 128
DV = D // 16

R0 = 74240

BLK = 1280
NBLK = R0 // BLK

NW = 32
NSC = N - R0
ROWS_W = NSC // NW
C = 128
NCHUNK = ROWS_W // C
ZR = NSEG // 16


def _sc_body(x_hbm, b2d_hbm, out_hbm, xbuf, ids, zbuf, shared, sems, ssems):
    cid = lax.axis_index("c")
    sid = lax.axis_index("s")
    wid = sid * 2 + cid
    base = R0 + wid * ROWS_W

    def dma_x(k, s):
        return pltpu.make_async_copy(
            x_hbm.at[pl.ds(base + k * C, C)], xbuf.at[s], sems.at[s]
        )

    def scat(k, s):
        return pltpu.make_async_copy(
            xbuf.at[s], shared.at[ids.at[k]], ssems.at[s]
        )

    idcp = pltpu.make_async_copy(b2d_hbm.at[wid], ids, sems.at[4])
    idcp.start()

    zero = jnp.zeros((16,), jnp.float32)

    def zrow(r, carry):
        row = zbuf.at[r]
        for j in range(DV):
            row[pl.ds(16 * j, 16)] = zero
        return carry

    lax.fori_loop(0, ZR, zrow, 0)
    pltpu.sync_copy(zbuf, shared.at[pl.ds(sid * ZR, ZR)])
    plsc.subcore_barrier()

    for s in range(4):
        dma_x(s, s).start()
    idcp.wait()

    def quad(q, carry):
        for s in range(4):
            k = 4 * q + s
            dma_x(k, s).wait()
            scat(k, s).start(add=True)
        for s in range(4):
            k = 4 * q + s
            scat(k, s).wait()

            @pl.when(k + 4 < NCHUNK)
            def _(k=k, s=s):
                dma_x(k + 4, s).start()

        return carry

    lax.fori_loop(0, NCHUNK // 4, quad, 0)

    plsc.subcore_barrier()
    pltpu.sync_copy(
        shared.at[pl.ds(sid * ZR, ZR)],
        out_hbm.at[cid].at[pl.ds(sid * ZR, ZR)],
    )


def _tc_body(batch_ref, x_ref, out_ref):
    i = pl.program_id(0)
    b = batch_ref[0, 0, :]
    onehot = (
        jax.lax.broadcasted_iota(jnp.int32, (NSEG, BLK), 0) == b[None, :]
    ).astype(jnp.float32)
    part = jax.lax.dot_general(
        onehot, x_ref[...], (((1,), (0,)), ((), ())),
        preferred_element_type=jnp.float32,
    )

    @pl.when(i == 0)
    def _():
        out_ref[...] = part

    @pl.when(i > 0)
    def _():
        out_ref[...] += part


def _combine_body(t_ref, p_ref, o_ref):
    o_ref[...] = t_ref[...] + p_ref[0] + p_ref[1]


def kernel(x, batch):
    b32 = batch.astype(jnp.int32)
    b2d = b32[R0:].reshape(NW, NCHUNK, C)
    batch3 = b32.reshape(N // BLK, 1, BLK)

    sc = pl.kernel(
        _sc_body,
        out_type=jax.ShapeDtypeStruct((2, NSEG, D), jnp.float32),
        mesh=plsc.VectorSubcoreMesh(core_axis_name="c", subcore_axis_name="s"),
        scratch_types=[
            pltpu.VMEM((4, C, D), jnp.float32),
            pltpu.VMEM((NCHUNK, C), jnp.int32),
            pltpu.VMEM((ZR, D), jnp.float32),
            pltpu.VMEM_SHARED((NSEG, D), jnp.float32),
            pltpu.SemaphoreType.DMA((5,)),
            pltpu.SemaphoreType.DMA((4,)),
        ],
    )
    partials = sc(x, b2d)

    tcp = pl.pallas_call(
        _tc_body,
        grid=(NBLK,),
        in_specs=[
            pl.BlockSpec((1, 1, BLK), lambda i: (i, 0, 0)),
            pl.BlockSpec((BLK, D), lambda i: (i, 0)),
        ],
        out_specs=pl.BlockSpec((NSEG, D), lambda i: (0, 0)),
        out_shape=jax.ShapeDtypeStruct((NSEG, D), jnp.float32),
    )(batch3, x)

    out = pl.pallas_call(
        _combine_body,
        out_shape=jax.ShapeDtypeStruct((NSEG, D), jnp.float32),
    )(tcp, partials)
    return out


# speedup vs baseline: 1.5559x; 1.0305x over previous
import functools

import jax
import jax.numpy as jnp
from jax import lax
from jax.experimental import pallas as pl
from jax.experimental.pallas import tpu as pltpu
from jax.experimental.pallas import tpu_sc as plsc

NSEG = 512
N = 320000
D = 128
DV = D // 16

R0 = 94720

BLK = 1280
NBLK = R0 // BLK

NW = 32
NSC = N - R0
ROWS_W = NSC // NW
C = 128
NCHUNK = ROWS_W // C
ZR = NSEG // 16


def _sc_body(x_hbm, b2d_hbm, out_hbm, xbuf, ids, zbuf, shared, sems, ssems):
    cid = lax.axis_index("c")
    sid = lax.axis_index("s")
    wid = sid * 2 + cid
    base = R0 + wid * ROWS_W

    def dma_x(k, s):
        return pltpu.make_async_copy(
            x_hbm.at[pl.ds(base + k * C, C)], xbuf.at[s], sems.at[s]
        )

    def scat(k, s):
        return pltpu.make_async_copy(
            xbuf.at[s], shared.at[ids.at[k]], ssems.at[s]
        )

    idcp = pltpu.make_async_copy(b2d_hbm.at[wid], ids, sems.at[4])
    idcp.start()

    zero = jnp.zeros((16,), jnp.float32)

    def zrow(r, carry):
        row = zbuf.at[r]
        for j in range(DV):
            row[pl.ds(16 * j, 16)] = zero
        return carry

    lax.fori_loop(0, ZR, zrow, 0)
    pltpu.sync_copy(zbuf, shared.at[pl.ds(sid * ZR, ZR)])
    plsc.subcore_barrier()

    for s in range(4):
        dma_x(s, s).start()
    idcp.wait()

    def quad(q, carry):
        for s in range(4):
            k = 4 * q + s
            dma_x(k, s).wait()
            scat(k, s).start(add=True)
        for s in range(4):
            k = 4 * q + s
            scat(k, s).wait()

            @pl.when(k + 4 < NCHUNK)
            def _(k=k, s=s):
                dma_x(k + 4, s).start()

        return carry

    lax.fori_loop(0, NCHUNK // 4, quad, 0)
    for r in range(NCHUNK % 4):
        k = (NCHUNK // 4) * 4 + r
        dma_x(k, r).wait()
        scat(k, r).start(add=True)
    for r in range(NCHUNK % 4):
        k = (NCHUNK // 4) * 4 + r
        scat(k, r).wait()

    plsc.subcore_barrier()
    pltpu.sync_copy(
        shared.at[pl.ds(sid * ZR, ZR)],
        out_hbm.at[cid].at[pl.ds(sid * ZR, ZR)],
    )


def _tc_body(batch_ref, x_ref, out_ref):
    i = pl.program_id(0)
    b = batch_ref[0, 0, :]
    onehot = (
        jax.lax.broadcasted_iota(jnp.int32, (NSEG, BLK), 0) == b[None, :]
    ).astype(jnp.float32)
    part = jax.lax.dot_general(
        onehot, x_ref[...], (((1,), (0,)), ((), ())),
        preferred_element_type=jnp.float32,
    )

    @pl.when(i == 0)
    def _():
        out_ref[...] = part

    @pl.when(i > 0)
    def _():
        out_ref[...] += part


def _combine_body(t_ref, p_ref, o_ref):
    o_ref[...] = t_ref[...] + p_ref[0] + p_ref[1]


def kernel(x, batch):
    b32 = batch.astype(jnp.int32)
    b2d = b32[R0:].reshape(NW, NCHUNK, C)
    batch3 = b32.reshape(N // BLK, 1, BLK)

    sc = pl.kernel(
        _sc_body,
        out_type=jax.ShapeDtypeStruct((2, NSEG, D), jnp.float32),
        mesh=plsc.VectorSubcoreMesh(core_axis_name="c", subcore_axis_name="s"),
        scratch_types=[
            pltpu.VMEM((4, C, D), jnp.float32),
            pltpu.VMEM((NCHUNK, C), jnp.int32),
            pltpu.VMEM((ZR, D), jnp.float32),
            pltpu.VMEM_SHARED((NSEG, D), jnp.float32),
            pltpu.SemaphoreType.DMA((5,)),
            pltpu.SemaphoreType.DMA((4,)),
        ],
    )
    partials = sc(x, b2d)

    tcp = pl.pallas_call(
        _tc_body,
        grid=(NBLK,),
        in_specs=[
            pl.BlockSpec((1, 1, BLK), lambda i: (i, 0, 0)),
            pl.BlockSpec((BLK, D), lambda i: (i, 0)),
        ],
        out_specs=pl.BlockSpec((NSEG, D), lambda i: (0, 0)),
        out_shape=jax.ShapeDtypeStruct((NSEG, D), jnp.float32),
    )(batch3, x)

    out = pl.pallas_call(
        _combine_body,
        out_shape=jax.ShapeDtypeStruct((NSEG, D), jnp.float32),
    )(tcp, partials)
    return out
